# baseline (speedup 1.0000x reference)
import functools

import jax
import jax.numpy as jnp
from jax import lax
from jax.experimental import pallas as pl
from jax.experimental.pallas import tpu as pltpu
from jax.experimental.pallas import tpu_sc as plsc

N = 10000
E = 320000
D = 128
NC = 2
NS = 16
NW = NC * NS
L = 16
CH = 80

EPW = E // NW
SB = 25
RT = 640
RT_LAST = N - 15 * RT

_mesh = plsc.VectorSubcoreMesh(
    core_axis_name="c", subcore_axis_name="s", num_cores=NC, num_subcores=NS)


@functools.partial(
    pl.kernel,
    out_type=jax.ShapeDtypeStruct((NC, N), jnp.float32),
    mesh=_mesh,
    scratch_types=[
        pltpu.VMEM((EPW // CH, CH), jnp.int32),
        pltpu.VMEM((N,), jnp.float32),
        pltpu.VMEM((CH,), jnp.float32),
        pltpu.VMEM_SHARED((N,), jnp.float32),
        pltpu.SemaphoreType.DMA,
        pltpu.SemaphoreType.DMA,
        pltpu.SemaphoreType.DMA,
        pltpu.SemaphoreType.DMA,
    ],
)
def _deg_kernel(dst3_hbm, init_hbm, ones_hbm, deg_hbm, idx_v, nbuf_v, ones_v,
                shared, dsem0, dsem1, dsem2, dsem3):
    c = lax.axis_index("c")
    s = lax.axis_index("s")
    wid = s * NC + c

    @pl.when(s == 0)
    def _():
        pltpu.sync_copy(init_hbm.at[c], nbuf_v)
        pltpu.sync_copy(nbuf_v, shared)

    pltpu.sync_copy(ones_hbm, ones_v)
    pltpu.sync_copy(dst3_hbm.at[wid], idx_v)
    plsc.subcore_barrier()

    dsems = (dsem0, dsem1, dsem2, dsem3)

    def body(i, _):
        ds = [pltpu.async_copy(ones_v, shared.at[idx_v.at[4 * i + k]],
                               dsems[k], add=True) for k in range(4)]
        for d in ds:
            d.wait()
        return 0

    lax.fori_loop(0, EPW // CH // 4, body, 0)
    pltpu.sync_copy(ones_v, shared.at[idx_v.at[EPW // CH - 1]], add=True)
    plsc.subcore_barrier()

    @pl.when(s == 0)
    def _():
        pltpu.sync_copy(shared, nbuf_v)
        pltpu.sync_copy(nbuf_v, deg_hbm.at[c])


def _mm_body(x_ref, w_ref, h_ref):
    h_ref[...] = jnp.dot(x_ref[...], w_ref[...],
                         preferred_element_type=jnp.float32)


def _matmul(x, W):
    bm = 1000
    return pl.pallas_call(
        _mm_body,
        grid=(N // bm,),
        in_specs=[
            pl.BlockSpec((bm, D), lambda i: (i, 0)),
            pl.BlockSpec((D, D), lambda i: (0, 0)),
        ],
        out_specs=pl.BlockSpec((bm, D), lambda i: (i, 0)),
        out_shape=jax.ShapeDtypeStruct((N, D), jnp.float32),
    )(x, W)


def _scale_body(h_ref, deg_ref, g_ref):
    deg = deg_ref[0] + deg_ref[1]
    g_ref[...] = h_ref[...] * lax.rsqrt(deg)


def _prescale(h, deg2):
    bm = 1000
    return pl.pallas_call(
        _scale_body,
        grid=(N // bm,),
        in_specs=[
            pl.BlockSpec((bm, D), lambda i: (i, 0)),
            pl.BlockSpec((NC, bm, 1), lambda i: (0, i, 0)),
        ],
        out_specs=pl.BlockSpec((bm, D), lambda i: (i, 0)),
        out_shape=jax.ShapeDtypeStruct((N, D), jnp.float32),
    )(h, deg2)


@functools.partial(
    pl.kernel,
    out_type=jax.ShapeDtypeStruct((NC, N, D), jnp.float32),
    mesh=_mesh,
    scratch_types=[
        pltpu.VMEM((SB, CH), jnp.int32),
        pltpu.VMEM((SB, CH), jnp.int32),
        pltpu.VMEM((CH, D), jnp.float32),
        pltpu.VMEM((CH, D), jnp.float32),
        pltpu.VMEM((CH, D), jnp.float32),
        pltpu.VMEM((CH, D), jnp.float32),
        pltpu.VMEM_SHARED((N, D), jnp.float32),
        pltpu.SemaphoreType.DMA,
        pltpu.SemaphoreType.DMA,
        pltpu.SemaphoreType.DMA,
        pltpu.SemaphoreType.DMA,
    ],
)
def _edge_kernel(g_hbm, src4_hbm, dst4_hbm, zero_hbm, agg_hbm,
                 sidx_v, didx_v, rows0_v, rows1_v, rows2_v, rows3_v, shared,
                 sem0, sem1, sem2, sem3):
    c = lax.axis_index("c")
    s = lax.axis_index("s")
    r0 = pl.multiple_of(s * RT, 8)
    nch = jnp.where(s == NS - 1, RT_LAST // CH, RT // CH)

    pltpu.sync_copy(zero_hbm, rows0_v)

    def init_body(j, _):
        rows = pl.ds(pl.multiple_of(r0 + j * CH, 8), CH)
        pltpu.sync_copy(rows0_v, shared.at[rows, :])
        return 0

    lax.fori_loop(0, nch, init_body, 0)

    wid = s * NC + c
    plsc.subcore_barrier()

    def gather(ch, buf, sem):
        pltpu.async_copy(g_hbm.at[sidx_v.at[ch]], buf, sem)

    def gwait(ch, buf, sem):
        pltpu.make_async_copy(g_hbm.at[sidx_v.at[ch]], buf, sem).wait()

    def scat(ch, buf):
        pltpu.sync_copy(buf, shared.at[didx_v.at[ch]], add=True)

    def sb_body(sb, _):
        pltpu.sync_copy(src4_hbm.at[wid].at[sb], sidx_v)
        pltpu.sync_copy(dst4_hbm.at[wid].at[sb], didx_v)
        def ag(ch, buf, sem):
            return pltpu.async_copy(g_hbm.at[sidx_v.at[ch]], buf, sem)

        def asc(ch, buf, sem):
            return pltpu.async_copy(buf, shared.at[didx_v.at[ch]], sem,
                                    add=True)

        d0 = ag(0, rows0_v, sem0)
        d1 = ag(1, rows1_v, sem1)
        d0.wait()
        d1.wait()

        def body(j, _):
            e0 = asc(4 * j, rows0_v, sem0)
            e1 = asc(4 * j + 1, rows1_v, sem1)
            d2 = ag(4 * j + 2, rows2_v, sem2)
            d3 = ag(4 * j + 3, rows3_v, sem3)
            e0.wait(); e1.wait(); d2.wait(); d3.wait()
            e2 = asc(4 * j + 2, rows2_v, sem2)
            e3 = asc(4 * j + 3, rows3_v, sem3)
            f0 = ag(4 * j + 4, rows0_v, sem0)
            f1 = ag(4 * j + 5, rows1_v, sem1)
            e2.wait(); e3.wait(); f0.wait(); f1.wait()
            return 0

        lax.fori_loop(0, 5, body, 0)
        e0 = asc(20, rows0_v, sem0)
        e1 = asc(21, rows1_v, sem1)
        d2 = ag(22, rows2_v, sem2)
        d3 = ag(23, rows3_v, sem3)
        e0.wait(); e1.wait(); d2.wait(); d3.wait()
        e2 = asc(22, rows2_v, sem2)
        e3 = asc(23, rows3_v, sem3)
        d0 = ag(24, rows0_v, sem0)
        e2.wait(); e3.wait(); d0.wait()
        scat(24, rows0_v)
        return 0

    lax.fori_loop(0, EPW // (SB * CH), sb_body, 0)
    plsc.subcore_barrier()

    def out_body(j, _):
        rows = pl.ds(pl.multiple_of(r0 + j * CH, 8), CH)
        pltpu.sync_copy(shared.at[rows, :], rows0_v)
        pltpu.sync_copy(rows0_v, agg_hbm.at[c].at[rows, :])
        return 0

    lax.fori_loop(0, nch, out_body, 0)


def _out_body(agg_ref, g_ref, deg_ref, b_ref, o_ref):
    deg = deg_ref[0] + deg_ref[1]
    dinv = lax.rsqrt(deg)
    acc = agg_ref[0] + agg_ref[1] + g_ref[...]
    o_ref[...] = jnp.maximum(acc * dinv + b_ref[...], 0.0)


def _combine(agg, g, deg2, b):
    bm = 1000
    return pl.pallas_call(
        _out_body,
        grid=(N // bm,),
        in_specs=[
            pl.BlockSpec((NC, bm, D), lambda i: (0, i, 0)),
            pl.BlockSpec((bm, D), lambda i: (i, 0)),
            pl.BlockSpec((NC, bm, 1), lambda i: (0, i, 0)),
            pl.BlockSpec((1, D), lambda i: (0, 0)),
        ],
        out_specs=pl.BlockSpec((bm, D), lambda i: (i, 0)),
        out_shape=jax.ShapeDtypeStruct((N, D), jnp.float32),
    )(agg, g, deg2, b.reshape(1, D))


def kernel(x, edge_index, W, b):
    src = edge_index[0].astype(jnp.int32)
    dst = edge_index[1].astype(jnp.int32)
    init = jnp.stack([jnp.ones((N,), jnp.float32),
                      jnp.zeros((N,), jnp.float32)])
    ones = jnp.ones((CH,), jnp.float32)
    zero = jnp.zeros((CH, D), jnp.float32)
    deg2 = _deg_kernel(dst.reshape(NW, EPW // CH, CH), init, ones)
    h = _matmul(x, W)
    g = _prescale(h, deg2.reshape(NC, N, 1))
    agg = _edge_kernel(g, src.reshape(NW, EPW // (SB * CH), SB, CH),
                       dst.reshape(NW, EPW // (SB * CH), SB, CH), zero)
    return _combine(agg, g, deg2.reshape(NC, N, 1), b)


# speedup vs baseline: 1.0257x; 1.0257x over previous
import functools

import jax
import jax.numpy as jnp
from jax import lax
from jax.experimental import pallas as pl
from jax.experimental.pallas import tpu as pltpu
from jax.experimental.pallas import tpu_sc as plsc

N = 10000
E = 320000
D = 128
NC = 2
NS = 16
NW = NC * NS
L = 16
CH = 80

EPW = E // NW
SB = 25
RT = 640
RT_LAST = N - 15 * RT

_mesh = plsc.VectorSubcoreMesh(
    core_axis_name="c", subcore_axis_name="s", num_cores=NC, num_subcores=NS)


@functools.partial(
    pl.kernel,
    out_type=jax.ShapeDtypeStruct((NC, N), jnp.float32),
    mesh=_mesh,
    scratch_types=[
        pltpu.VMEM((EPW // CH, CH), jnp.int32),
        pltpu.VMEM((N,), jnp.float32),
        pltpu.VMEM((CH,), jnp.float32),
        pltpu.VMEM_SHARED((N,), jnp.float32),
        pltpu.SemaphoreType.DMA,
        pltpu.SemaphoreType.DMA,
        pltpu.SemaphoreType.DMA,
        pltpu.SemaphoreType.DMA,
    ],
)
def _deg_kernel(dst3_hbm, init_hbm, ones_hbm, deg_hbm, idx_v, nbuf_v, ones_v,
                shared, dsem0, dsem1, dsem2, dsem3):
    c = lax.axis_index("c")
    s = lax.axis_index("s")
    wid = s * NC + c

    @pl.when(s == 0)
    def _():
        pltpu.sync_copy(init_hbm.at[c], nbuf_v)
        pltpu.sync_copy(nbuf_v, shared)

    pltpu.sync_copy(ones_hbm, ones_v)
    pltpu.sync_copy(dst3_hbm.at[wid], idx_v)
    plsc.subcore_barrier()

    dsems = (dsem0, dsem1, dsem2, dsem3)

    def body(i, _):
        ds = [pltpu.async_copy(ones_v, shared.at[idx_v.at[4 * i + k]],
                               dsems[k], add=True) for k in range(4)]
        for d in ds:
            d.wait()
        return 0

    lax.fori_loop(0, EPW // CH // 4, body, 0)
    pltpu.sync_copy(ones_v, shared.at[idx_v.at[EPW // CH - 1]], add=True)
    plsc.subcore_barrier()

    @pl.when(s == 0)
    def _():
        pltpu.sync_copy(shared, nbuf_v)
        pltpu.sync_copy(nbuf_v, deg_hbm.at[c])


def _mm_body(x_ref, w_ref, deg_ref, g_ref):
    h = jnp.dot(x_ref[...], w_ref[...], preferred_element_type=jnp.float32)
    deg = deg_ref[0] + deg_ref[1]
    g_ref[...] = h * lax.rsqrt(deg)


def _matmul_scale(x, W, deg2):
    bm = 1000
    return pl.pallas_call(
        _mm_body,
        grid=(N // bm,),
        in_specs=[
            pl.BlockSpec((bm, D), lambda i: (i, 0)),
            pl.BlockSpec((D, D), lambda i: (0, 0)),
            pl.BlockSpec((NC, bm, 1), lambda i: (0, i, 0)),
        ],
        out_specs=pl.BlockSpec((bm, D), lambda i: (i, 0)),
        out_shape=jax.ShapeDtypeStruct((N, D), jnp.float32),
    )(x, W, deg2)


@functools.partial(
    pl.kernel,
    out_type=jax.ShapeDtypeStruct((NC, N, D), jnp.float32),
    mesh=_mesh,
    scratch_types=[
        pltpu.VMEM((SB, CH), jnp.int32),
        pltpu.VMEM((SB, CH), jnp.int32),
        pltpu.VMEM((CH, D), jnp.float32),
        pltpu.VMEM((CH, D), jnp.float32),
        pltpu.VMEM((CH, D), jnp.float32),
        pltpu.VMEM((CH, D), jnp.float32),
        pltpu.VMEM_SHARED((N, D), jnp.float32),
        pltpu.SemaphoreType.DMA,
        pltpu.SemaphoreType.DMA,
        pltpu.SemaphoreType.DMA,
        pltpu.SemaphoreType.DMA,
    ],
)
def _edge_kernel(g_hbm, src4_hbm, dst4_hbm, zero_hbm, agg_hbm,
                 sidx_v, didx_v, rows0_v, rows1_v, rows2_v, rows3_v, shared,
                 sem0, sem1, sem2, sem3):
    c = lax.axis_index("c")
    s = lax.axis_index("s")
    r0 = pl.multiple_of(s * RT, 8)
    nch = jnp.where(s == NS - 1, RT_LAST // CH, RT // CH)

    pltpu.sync_copy(zero_hbm, rows0_v)

    def init_body(j, _):
        rows = pl.ds(pl.multiple_of(r0 + j * CH, 8), CH)
        pltpu.sync_copy(rows0_v, shared.at[rows, :])
        return 0

    lax.fori_loop(0, nch, init_body, 0)

    wid = s * NC + c
    plsc.subcore_barrier()

    def gather(ch, buf, sem):
        pltpu.async_copy(g_hbm.at[sidx_v.at[ch]], buf, sem)

    def gwait(ch, buf, sem):
        pltpu.make_async_copy(g_hbm.at[sidx_v.at[ch]], buf, sem).wait()

    def scat(ch, buf):
        pltpu.sync_copy(buf, shared.at[didx_v.at[ch]], add=True)

    def sb_body(sb, _):
        pltpu.sync_copy(src4_hbm.at[wid].at[sb], sidx_v)
        di = pltpu.async_copy(dst4_hbm.at[wid].at[sb], didx_v, sem2)
        def ag(ch, buf, sem):
            return pltpu.async_copy(g_hbm.at[sidx_v.at[ch]], buf, sem)

        def asc(ch, buf, sem):
            return pltpu.async_copy(buf, shared.at[didx_v.at[ch]], sem,
                                    add=True)

        d0 = ag(0, rows0_v, sem0)
        d1 = ag(1, rows1_v, sem1)
        d0.wait()
        d1.wait()
        di.wait()

        def body(j, _):
            e0 = asc(4 * j, rows0_v, sem0)
            e1 = asc(4 * j + 1, rows1_v, sem1)
            d2 = ag(4 * j + 2, rows2_v, sem2)
            d3 = ag(4 * j + 3, rows3_v, sem3)
            e0.wait(); e1.wait(); d2.wait(); d3.wait()
            e2 = asc(4 * j + 2, rows2_v, sem2)
            e3 = asc(4 * j + 3, rows3_v, sem3)
            f0 = ag(4 * j + 4, rows0_v, sem0)
            f1 = ag(4 * j + 5, rows1_v, sem1)
            e2.wait(); e3.wait(); f0.wait(); f1.wait()
            return 0

        lax.fori_loop(0, 5, body, 0)
        e0 = asc(20, rows0_v, sem0)
        e1 = asc(21, rows1_v, sem1)
        d2 = ag(22, rows2_v, sem2)
        d3 = ag(23, rows3_v, sem3)
        e0.wait(); e1.wait(); d2.wait(); d3.wait()
        e2 = asc(22, rows2_v, sem2)
        e3 = asc(23, rows3_v, sem3)
        d0 = ag(24, rows0_v, sem0)
        e2.wait(); e3.wait(); d0.wait()
        scat(24, rows0_v)
        return 0

    lax.fori_loop(0, EPW // (SB * CH), sb_body, 0)
    plsc.subcore_barrier()

    def out_body(j, _):
        rows = pl.ds(pl.multiple_of(r0 + j * CH, 8), CH)
        pltpu.sync_copy(shared.at[rows, :], rows0_v)
        pltpu.sync_copy(rows0_v, agg_hbm.at[c].at[rows, :])
        return 0

    lax.fori_loop(0, nch, out_body, 0)


def _out_body(agg_ref, g_ref, deg_ref, b_ref, o_ref):
    deg = deg_ref[0] + deg_ref[1]
    dinv = lax.rsqrt(deg)
    acc = agg_ref[0] + agg_ref[1] + g_ref[...]
    o_ref[...] = jnp.maximum(acc * dinv + b_ref[...], 0.0)


def _combine(agg, g, deg2, b):
    bm = 1000
    return pl.pallas_call(
        _out_body,
        grid=(N // bm,),
        in_specs=[
            pl.BlockSpec((NC, bm, D), lambda i: (0, i, 0)),
            pl.BlockSpec((bm, D), lambda i: (i, 0)),
            pl.BlockSpec((NC, bm, 1), lambda i: (0, i, 0)),
            pl.BlockSpec((1, D), lambda i: (0, 0)),
        ],
        out_specs=pl.BlockSpec((bm, D), lambda i: (i, 0)),
        out_shape=jax.ShapeDtypeStruct((N, D), jnp.float32),
    )(agg, g, deg2, b.reshape(1, D))


def kernel(x, edge_index, W, b):
    src = edge_index[0].astype(jnp.int32)
    dst = edge_index[1].astype(jnp.int32)
    init = jnp.stack([jnp.ones((N,), jnp.float32),
                      jnp.zeros((N,), jnp.float32)])
    ones = jnp.ones((CH,), jnp.float32)
    zero = jnp.zeros((CH, D), jnp.float32)
    deg2 = _deg_kernel(dst.reshape(NW, EPW // CH, CH), init, ones)
    g = _matmul_scale(x, W, deg2.reshape(NC, N, 1))
    agg = _edge_kernel(g, src.reshape(NW, EPW // (SB * CH), SB, CH),
                       dst.reshape(NW, EPW // (SB * CH), SB, CH), zero)
    return _combine(agg, g, deg2.reshape(NC, N, 1), b)
